# baseline (speedup 1.0000x reference)
import functools

import jax
import jax.numpy as jnp
import numpy as np
from jax.experimental import pallas as pl
from jax.experimental.pallas import tpu as pltpu

E = 8
D = 768
FF = 768
ALPHA = 1.702
LIMIT = 7.0

_SEL_NP = np.zeros((2 * FF, FF), dtype=np.float32)
_SEL_NP[::2, :] = np.eye(FF, dtype=np.float32)
_SEL = _SEL_NP.astype(jnp.bfloat16)


def _moe_body(hs_ref, rwt_ref, rb_ref, guw_ref, gub_ref, dw_ref, db_ref,
              sel_ref, out_ref, scores_ref, wb_s, dw2_s, *, bt):
    e = pl.program_id(0)
    t = pl.program_id(1)
    x = hs_ref[...]

    @pl.when(t == 0)
    def _prep_weights():
        wb_s[...] = guw_ref[0].astype(jnp.bfloat16)
        dw2_s[...] = jnp.dot(sel_ref[...], dw_ref[0].astype(jnp.bfloat16),
                             preferred_element_type=jnp.float32
                             ).astype(jnp.bfloat16)

    @pl.when(e == 0)
    def _router():
        logits = jnp.dot(x, rwt_ref[...], preferred_element_type=jnp.float32)
        logits = logits + rb_ref[...]
        col = jax.lax.broadcasted_iota(jnp.int32, logits.shape, 1)
        m1 = jnp.max(logits, axis=1, keepdims=True)
        a1 = jnp.min(jnp.where(logits == m1, col, E), axis=1, keepdims=True)
        rest = jnp.where(col == a1, -jnp.inf, logits)
        m2 = jnp.max(rest, axis=1, keepdims=True)
        a2 = jnp.min(jnp.where(rest == m2, col, E), axis=1, keepdims=True)
        p1 = 1.0 / (1.0 + jnp.exp(m2 - m1))
        p2 = 1.0 - p1
        scores = jnp.where(col == a1, p1, jnp.where(col == a2, p2, 0.0))
        scores_ref[pl.ds(t * bt, bt), :] = scores

    scores_blk = scores_ref[pl.ds(t * bt, bt), :]
    col = jax.lax.broadcasted_iota(jnp.int32, scores_blk.shape, 1)
    w = jnp.sum(jnp.where(col == e, scores_blk, 0.0), axis=1, keepdims=True)

    xb = x.astype(jnp.bfloat16)
    gu = jnp.dot(xb, wb_s[...], preferred_element_type=jnp.float32) + gub_ref[0]
    gu_r = pltpu.roll(gu, 2 * FF - 1, 1)
    g = jnp.minimum(gu, LIMIT)
    u = jnp.clip(gu_r, -LIMIT, LIMIT)
    glu = g / (1.0 + jnp.exp(-ALPHA * g))
    act2 = ((u + 1.0) * glu).astype(jnp.bfloat16)
    contrib = jnp.dot(act2, dw2_s[...], preferred_element_type=jnp.float32)
    contrib = w * (contrib + db_ref[0])

    sl = pl.ds(t * bt, bt)

    @pl.when(e == 0)
    def _init():
        out_ref[sl, :] = contrib

    @pl.when(e != 0)
    def _acc():
        out_ref[sl, :] = out_ref[sl, :] + contrib


def kernel(hidden_states, router_weight, router_bias, gate_up_proj,
           gate_up_proj_bias, down_proj, down_proj_bias):
    bsz, seq, d = hidden_states.shape
    T = bsz * seq
    hs = hidden_states.reshape(T, d)
    BT = 1024
    NT = T // BT

    rwt = router_weight.T
    rb = router_bias.reshape(1, E)
    gub = gate_up_proj_bias.reshape(E, 1, 2 * FF)
    db = down_proj_bias.reshape(E, 1, D)
    sel = _SEL

    grid = (E, NT)
    out, scores = pl.pallas_call(
        functools.partial(_moe_body, bt=BT),
        grid=grid,
        in_specs=[
            pl.BlockSpec((BT, D), lambda e, t: (t, 0)),
            pl.BlockSpec((D, E), lambda e, t: (0, 0)),
            pl.BlockSpec((1, E), lambda e, t: (0, 0)),
            pl.BlockSpec((1, D, 2 * FF), lambda e, t: (e, 0, 0)),
            pl.BlockSpec((1, 1, 2 * FF), lambda e, t: (e, 0, 0)),
            pl.BlockSpec((1, FF, D), lambda e, t: (e, 0, 0)),
            pl.BlockSpec((1, 1, D), lambda e, t: (e, 0, 0)),
            pl.BlockSpec((2 * FF, FF), lambda e, t: (0, 0)),
        ],
        out_specs=[
            pl.BlockSpec((T, D), lambda e, t: (0, 0)),
            pl.BlockSpec((T, E), lambda e, t: (0, 0)),
        ],
        out_shape=[
            jax.ShapeDtypeStruct((T, D), jnp.float32),
            jax.ShapeDtypeStruct((T, E), jnp.float32),
        ],
        scratch_shapes=[
            pltpu.VMEM((D, 2 * FF), jnp.bfloat16),
            pltpu.VMEM((2 * FF, D), jnp.bfloat16),
        ],
        compiler_params=pltpu.CompilerParams(
            dimension_semantics=("arbitrary", "arbitrary"),
        ),
    )(hs, rwt, rb, gate_up_proj, gub, down_proj, db, sel)

    return out.reshape(bsz, seq, d), scores
